# baseline (speedup 1.0000x reference)
import functools

import jax
import jax.numpy as jnp
from jax import lax
from jax.experimental import pallas as pl
from jax.experimental.pallas import tpu as pltpu
from jax.experimental.pallas import tpu_sc as plsc

BATCH = 4096
SEQ_LEN = 200
EMB = 64

NUM_CORES = 2
NUM_SUBCORES = 16
NUM_WORKERS = NUM_CORES * NUM_SUBCORES

TOTAL = BATCH * SEQ_LEN
PER_WORKER = TOTAL // NUM_WORKERS
CHUNK = 128
CHUNKS_PER_WORKER = PER_WORKER // CHUNK
BG = BATCH // CHUNK
NBUF = 4
PREFETCH = 2

_LANE = 16
_CG = EMB // 8


def _make_kernel():
    mesh = plsc.VectorSubcoreMesh(core_axis_name="c", subcore_axis_name="s")

    @functools.partial(
        pl.kernel,
        out_type=jax.ShapeDtypeStruct((SEQ_LEN, _CG, BG, 8, CHUNK), jnp.float32),
        mesh=mesh,
        scratch_types=[
            pltpu.VMEM((PER_WORKER,), jnp.int32),
            pltpu.VMEM((NBUF, CHUNK, EMB), jnp.float32),
            pltpu.VMEM((NBUF, _CG, 8, CHUNK), jnp.float32),
            pltpu.VMEM((SEQ_LEN, EMB), jnp.float32),
            [pltpu.SemaphoreType.DMA] * NBUF,
            [pltpu.SemaphoreType.DMA] * NBUF,
        ],
        compiler_params=pltpu.CompilerParams(
            use_tc_tiling_on_sc=False, needs_layout_passes=False),
    )
    def emb_kernel(idx_hbm, wt_hbm, pos_hbm, out_hbm,
                   idx_v, rows_v, outt_v, pos_v, g_sems, o_sems):
        wid = lax.axis_index("s") * NUM_CORES + lax.axis_index("c")
        base = wid * PER_WORKER
        kbase = wid * CHUNKS_PER_WORKER

        pltpu.sync_copy(idx_hbm.at[pl.ds(base, PER_WORKER)], idx_v)
        pltpu.sync_copy(pos_hbm, pos_v)

        def start_gather(g, b):
            pltpu.async_copy(wt_hbm.at[idx_v.at[pl.ds(g * CHUNK, CHUNK)]],
                             rows_v.at[b], g_sems[b])

        for b in range(PREFETCH):
            start_gather(b, b)

        @pl.loop(0, CHUNKS_PER_WORKER, step=NBUF)
        def _block(k):
            for b in range(NBUF):
                g = k + b
                kk = kbase + g
                l = kk // BG
                bg = lax.rem(kk, BG)

                pltpu.make_async_copy(
                    wt_hbm.at[idx_v.at[pl.ds(g * CHUNK, CHUNK)]],
                    rows_v.at[b], g_sems[b]).wait()

                @pl.loop(0, CHUNK)
                def _row(j):
                    for s in range(4):
                        sl = pl.ds(s * _LANE, _LANE)
                        rows_v[b, j, sl] = rows_v[b, j, sl] + pos_v[l, sl]

                @pl.when(g >= NBUF)
                def _():
                    gp = g - NBUF
                    kp = kbase + gp
                    pltpu.make_async_copy(
                        outt_v.at[b],
                        out_hbm.at[kp // BG, :, lax.rem(kp, BG)],
                        o_sems[b]).wait()

                rows2d = rows_v.at[b]
                outt2d = outt_v.at[b]

                @pl.loop(0, EMB)
                def _col(c):
                    iota = lax.iota(jnp.int32, _LANE)
                    cidx = jnp.full((_LANE,), c, dtype=jnp.int32)
                    for v in range(CHUNK // _LANE):
                        vec = plsc.load_gather(rows2d, [iota + v * _LANE, cidx])
                        outt2d[c // 8, lax.rem(c, 8),
                               pl.ds(v * _LANE, _LANE)] = vec

                pltpu.async_copy(outt_v.at[b], out_hbm.at[l, :, bg], o_sems[b])

                t = g + PREFETCH

                @pl.when(t < CHUNKS_PER_WORKER)
                def _():
                    start_gather(t, (b + PREFETCH) % NBUF)

        for g in range(CHUNKS_PER_WORKER - NBUF, CHUNKS_PER_WORKER):
            b = g % NBUF
            kk = kbase + g
            pltpu.make_async_copy(outt_v.at[b],
                                  out_hbm.at[kk // BG, :, lax.rem(kk, BG)],
                                  o_sems[b]).wait()

    return emb_kernel


def kernel(inputs, word_table, pos_table):
    idx_t_flat = inputs.T.reshape(TOTAL).astype(jnp.int32)
    out5d = _make_kernel()(idx_t_flat, word_table, pos_table)
    return out5d.transpose((2, 4, 0, 1, 3)).reshape(BATCH, SEQ_LEN, EMB)


# speedup vs baseline: 1.9632x; 1.9632x over previous
import functools

import jax
import jax.numpy as jnp
from jax import lax
from jax.experimental import pallas as pl
from jax.experimental.pallas import tpu as pltpu
from jax.experimental.pallas import tpu_sc as plsc

BATCH = 4096
SEQ_LEN = 200
EMB = 64

NUM_CORES = 2
NUM_SUBCORES = 16
NUM_WORKERS = NUM_CORES * NUM_SUBCORES

TOTAL = BATCH * SEQ_LEN
PER_WORKER = TOTAL // NUM_WORKERS
CHUNK = 128
CHUNKS_PER_WORKER = PER_WORKER // CHUNK
BG = BATCH // CHUNK
NBUF = 4
PREFETCH = 2

_LANE = 16
_CG = EMB // 8
_PITCH = CHUNK + 1


def _make_kernel():
    mesh = plsc.VectorSubcoreMesh(core_axis_name="c", subcore_axis_name="s")

    @functools.partial(
        pl.kernel,
        out_type=jax.ShapeDtypeStruct((SEQ_LEN, _CG, BG, 8, CHUNK), jnp.float32),
        mesh=mesh,
        scratch_types=[
            pltpu.VMEM((PER_WORKER,), jnp.int32),
            pltpu.VMEM((NBUF, CHUNK, EMB), jnp.float32),
            pltpu.VMEM((NBUF, EMB, _PITCH), jnp.float32),
            pltpu.VMEM((SEQ_LEN, EMB), jnp.float32),
            [pltpu.SemaphoreType.DMA] * NBUF,
            [pltpu.SemaphoreType.DMA] * NBUF,
        ],
        compiler_params=pltpu.CompilerParams(
            use_tc_tiling_on_sc=False, needs_layout_passes=False),
    )
    def emb_kernel(idx_hbm, wt_hbm, pos_hbm, out_hbm,
                   idx_v, rows_v, outt_v, pos_v, g_sems, o_sems):
        wid = lax.axis_index("s") * NUM_CORES + lax.axis_index("c")
        base = wid * PER_WORKER
        kbase = wid * CHUNKS_PER_WORKER

        pltpu.sync_copy(idx_hbm.at[pl.ds(base, PER_WORKER)], idx_v)
        pltpu.sync_copy(pos_hbm, pos_v)

        def start_gather(g, b):
            pltpu.async_copy(wt_hbm.at[idx_v.at[pl.ds(g * CHUNK, CHUNK)]],
                             rows_v.at[b], g_sems[b])

        def writeback(b, l, bg, issue):
            for cg in range(_CG):
                cp = pltpu.make_async_copy(
                    outt_v.at[b, pl.ds(cg * 8, 8), pl.ds(0, CHUNK)],
                    out_hbm.at[l, cg, bg], o_sems[b])
                if issue:
                    cp.start()
                else:
                    cp.wait()

        for b in range(PREFETCH):
            start_gather(b, b)

        @pl.loop(0, CHUNKS_PER_WORKER, step=NBUF)
        def _block(k):
            for b in range(NBUF):
                g = k + b
                kk = kbase + g
                l = kk // BG
                bg = lax.rem(kk, BG)

                pltpu.make_async_copy(
                    wt_hbm.at[idx_v.at[pl.ds(g * CHUNK, CHUNK)]],
                    rows_v.at[b], g_sems[b]).wait()

                @pl.when(g >= NBUF)
                def _():
                    kp = kbase + g - NBUF
                    writeback(b, kp // BG, lax.rem(kp, BG), issue=False)

                outt2d = outt_v.at[b]

                @pl.loop(0, CHUNK)
                def _row(j):
                    iota = lax.iota(jnp.int32, _LANE)
                    jidx = jnp.full((_LANE,), j, dtype=jnp.int32)
                    for s in range(4):
                        sl = pl.ds(s * _LANE, _LANE)
                        vec = rows_v[b, j, sl] + pos_v[l, sl]
                        plsc.store_scatter(outt2d,
                                           [iota + s * _LANE, jidx], vec)

                writeback(b, l, bg, issue=True)

                t = g + PREFETCH

                @pl.when(t < CHUNKS_PER_WORKER)
                def _():
                    start_gather(t, (b + PREFETCH) % NBUF)

        for g in range(CHUNKS_PER_WORKER - NBUF, CHUNKS_PER_WORKER):
            b = g % NBUF
            kk = kbase + g
            writeback(b, kk // BG, lax.rem(kk, BG), issue=False)

    return emb_kernel


def kernel(inputs, word_table, pos_table):
    idx_t_flat = inputs.T.reshape(TOTAL).astype(jnp.int32)
    out5d = _make_kernel()(idx_t_flat, word_table, pos_table)
    return out5d.transpose((2, 4, 0, 1, 3)).reshape(BATCH, SEQ_LEN, EMB)


# speedup vs baseline: 2.0509x; 1.0447x over previous
import functools

import jax
import jax.numpy as jnp
from jax import lax
from jax.experimental import pallas as pl
from jax.experimental.pallas import tpu as pltpu
from jax.experimental.pallas import tpu_sc as plsc

BATCH = 4096
SEQ_LEN = 200
EMB = 64

NUM_CORES = 2
NUM_SUBCORES = 16
NUM_WORKERS = NUM_CORES * NUM_SUBCORES

TOTAL = BATCH * SEQ_LEN
PER_WORKER = TOTAL // NUM_WORKERS
CHUNK = 128
CHUNKS_PER_WORKER = PER_WORKER // CHUNK
BG = BATCH // CHUNK
NBUF = 4
PREFETCH = 2

_LANE = 16
_CG = EMB // 8
_PITCH = CHUNK + 1


def _make_kernel():
    mesh = plsc.VectorSubcoreMesh(core_axis_name="c", subcore_axis_name="s")

    @functools.partial(
        pl.kernel,
        out_type=jax.ShapeDtypeStruct((SEQ_LEN, _CG, BG, 8, CHUNK), jnp.float32),
        mesh=mesh,
        scratch_types=[
            pltpu.VMEM((PER_WORKER,), jnp.int32),
            pltpu.VMEM((NBUF, CHUNK, EMB), jnp.float32),
            pltpu.VMEM((NBUF, EMB, _PITCH), jnp.float32),
            pltpu.VMEM((SEQ_LEN, EMB), jnp.float32),
            [pltpu.SemaphoreType.DMA] * NBUF,
            [pltpu.SemaphoreType.DMA] * NBUF,
        ],
        compiler_params=pltpu.CompilerParams(
            use_tc_tiling_on_sc=False, needs_layout_passes=False),
    )
    def emb_kernel(idx_hbm, wt_hbm, pos_hbm, out_hbm,
                   idx_v, rows_v, outt_v, pos_v, g_sems, o_sems):
        wid = lax.axis_index("s") * NUM_CORES + lax.axis_index("c")
        base = wid * PER_WORKER
        kbase = wid * CHUNKS_PER_WORKER

        pltpu.sync_copy(idx_hbm.at[pl.ds(base, PER_WORKER)], idx_v)
        pltpu.sync_copy(pos_hbm, pos_v)

        def start_gather(g, b):
            pltpu.async_copy(wt_hbm.at[idx_v.at[pl.ds(g * CHUNK, CHUNK)]],
                             rows_v.at[b], g_sems[b])

        def writeback(b, l, bg, issue):
            for cg in range(_CG):
                cp = pltpu.make_async_copy(
                    outt_v.at[b, pl.ds(cg * 8, 8), pl.ds(0, CHUNK)],
                    out_hbm.at[l, cg, bg], o_sems[b])
                if issue:
                    cp.start()
                else:
                    cp.wait()

        for b in range(PREFETCH):
            start_gather(b, b)

        @pl.loop(0, CHUNKS_PER_WORKER, step=NBUF)
        def _block(k):
            for b in range(NBUF):
                g = k + b
                kk = kbase + g
                l = kk // BG
                bg = lax.rem(kk, BG)

                pltpu.make_async_copy(
                    wt_hbm.at[idx_v.at[pl.ds(g * CHUNK, CHUNK)]],
                    rows_v.at[b], g_sems[b]).wait()

                @pl.when(g >= NBUF)
                def _():
                    kp = kbase + g - NBUF
                    writeback(b, kp // BG, lax.rem(kp, BG), issue=False)

                outt2d = outt_v.at[b]
                iota = lax.iota(jnp.int32, _LANE)
                pvs = [pos_v[l, pl.ds(s * _LANE, _LANE)] for s in range(4)]

                @pl.loop(0, CHUNK, unroll=4)
                def _row(j):
                    jidx = jnp.full((_LANE,), j, dtype=jnp.int32)
                    for s in range(4):
                        sl = pl.ds(s * _LANE, _LANE)
                        vec = rows_v[b, j, sl] + pvs[s]
                        plsc.store_scatter(outt2d,
                                           [iota + s * _LANE, jidx], vec)

                writeback(b, l, bg, issue=True)

                t = g + PREFETCH

                @pl.when(t < CHUNKS_PER_WORKER)
                def _():
                    start_gather(t, (b + PREFETCH) % NBUF)

        for g in range(CHUNKS_PER_WORKER - NBUF, CHUNKS_PER_WORKER):
            b = g % NBUF
            kk = kbase + g
            writeback(b, kk // BG, lax.rem(kk, BG), issue=False)

    return emb_kernel


def kernel(inputs, word_table, pos_table):
    idx_t_flat = inputs.T.reshape(TOTAL).astype(jnp.int32)
    out5d = _make_kernel()(idx_t_flat, word_table, pos_table)
    return out5d.transpose((2, 4, 0, 1, 3)).reshape(BATCH, SEQ_LEN, EMB)


# speedup vs baseline: 2.1724x; 1.0592x over previous
import functools

import jax
import jax.numpy as jnp
from jax import lax
from jax.experimental import pallas as pl
from jax.experimental.pallas import tpu as pltpu
from jax.experimental.pallas import tpu_sc as plsc

BATCH = 4096
SEQ_LEN = 200
EMB = 64

NUM_CORES = 2
NUM_SUBCORES = 16
NUM_WORKERS = NUM_CORES * NUM_SUBCORES

TOTAL = BATCH * SEQ_LEN
PER_WORKER = TOTAL // NUM_WORKERS
CHUNK = 128
CHUNKS_PER_WORKER = PER_WORKER // CHUNK
BG = BATCH // CHUNK
NBUF = 2
PREFETCH = 2

_LANE = 16
_CG = EMB // 8
_PITCH = CHUNK + 1


def _make_kernel():
    mesh = plsc.VectorSubcoreMesh(core_axis_name="c", subcore_axis_name="s")

    @functools.partial(
        pl.kernel,
        out_type=jax.ShapeDtypeStruct((SEQ_LEN, _CG, BG, 8, CHUNK), jnp.float32),
        mesh=mesh,
        scratch_types=[
            pltpu.VMEM((PER_WORKER,), jnp.int32),
            pltpu.VMEM((NBUF, CHUNK, 2 * EMB), jnp.float32),
            pltpu.VMEM((NBUF, EMB, _PITCH), jnp.float32),
            pltpu.VMEM((SEQ_LEN, EMB), jnp.float32),
            [pltpu.SemaphoreType.DMA] * NBUF,
            [pltpu.SemaphoreType.DMA] * NBUF,
        ],
        compiler_params=pltpu.CompilerParams(
            use_tc_tiling_on_sc=False, needs_layout_passes=False),
    )
    def emb_kernel(idx_hbm, wt_hbm, pos_hbm, out_hbm,
                   idx_v, rows_v, outt_v, pos_v, g_sems, o_sems):
        wid = lax.axis_index("s") * NUM_CORES + lax.axis_index("c")
        base = wid * PER_WORKER
        kbase = wid * CHUNKS_PER_WORKER

        pltpu.sync_copy(idx_hbm.at[pl.ds(base, PER_WORKER)], idx_v)
        pltpu.sync_copy(pos_hbm, pos_v)

        def start_gather(g, b):
            pltpu.async_copy(wt_hbm.at[idx_v.at[pl.ds(g * CHUNK, CHUNK)]],
                             rows_v.at[b], g_sems[b])

        def writeback(b, l, bg, issue):
            for cg in range(_CG):
                cp = pltpu.make_async_copy(
                    outt_v.at[b, pl.ds(cg * 8, 8), pl.ds(0, CHUNK)],
                    out_hbm.at[l, cg, bg], o_sems[b])
                if issue:
                    cp.start()
                else:
                    cp.wait()

        for b in range(PREFETCH):
            start_gather(b, b)

        @pl.loop(0, CHUNKS_PER_WORKER, step=NBUF)
        def _block(k):
            for b in range(NBUF):
                g = k + b
                kk = kbase + g
                l = kk // BG
                bg = lax.rem(kk, BG)

                pltpu.make_async_copy(
                    wt_hbm.at[idx_v.at[pl.ds(g * CHUNK, CHUNK)]],
                    rows_v.at[b], g_sems[b]).wait()

                @pl.when(g >= NBUF)
                def _():
                    kp = kbase + g - NBUF
                    writeback(b, kp // BG, lax.rem(kp, BG), issue=False)

                outt2d = outt_v.at[b]
                iota = lax.iota(jnp.int32, _LANE)
                pvs = [pos_v[l, pl.ds(s * _LANE, _LANE)] for s in range(4)]

                @pl.loop(0, CHUNK, unroll=4)
                def _row(j):
                    jidx = jnp.full((_LANE,), j, dtype=jnp.int32)
                    for s in range(4):
                        sl = pl.ds(s * _LANE, _LANE)
                        vec = rows_v[b, j, sl] + pvs[s]
                        plsc.store_scatter(outt2d,
                                           [iota + s * _LANE, jidx], vec)

                writeback(b, l, bg, issue=True)

                t = g + PREFETCH

                @pl.when(t < CHUNKS_PER_WORKER)
                def _():
                    start_gather(t, (b + PREFETCH) % NBUF)

        for g in range(CHUNKS_PER_WORKER - NBUF, CHUNKS_PER_WORKER):
            b = g % NBUF
            kk = kbase + g
            writeback(b, kk // BG, lax.rem(kk, BG), issue=False)

    return emb_kernel


def kernel(inputs, word_table, pos_table):
    idx_t_flat = inputs.T.reshape(TOTAL).astype(jnp.int32)
    wt_pad = jnp.pad(word_table, ((0, 0), (0, EMB)))
    out5d = _make_kernel()(idx_t_flat, wt_pad, pos_table)
    return out5d.transpose((2, 4, 0, 1, 3)).reshape(BATCH, SEQ_LEN, EMB)


# speedup vs baseline: 2.1846x; 1.0056x over previous
import functools

import jax
import jax.numpy as jnp
from jax import lax
from jax.experimental import pallas as pl
from jax.experimental.pallas import tpu as pltpu
from jax.experimental.pallas import tpu_sc as plsc

BATCH = 4096
SEQ_LEN = 200
EMB = 64

NUM_CORES = 2
NUM_SUBCORES = 16
NUM_WORKERS = NUM_CORES * NUM_SUBCORES

TOTAL = BATCH * SEQ_LEN
PER_WORKER = TOTAL // NUM_WORKERS
CHUNK = 128
CHUNKS_PER_WORKER = PER_WORKER // CHUNK
BG = BATCH // CHUNK
NBUF = 2
PREFETCH = 2

_LANE = 16
_CG = EMB // 8
_PITCH = CHUNK + 1


def _make_kernel():
    mesh = plsc.VectorSubcoreMesh(core_axis_name="c", subcore_axis_name="s")

    @functools.partial(
        pl.kernel,
        out_type=jax.ShapeDtypeStruct((SEQ_LEN, _CG, BG, 8, CHUNK), jnp.float32),
        mesh=mesh,
        scratch_types=[
            pltpu.VMEM((PER_WORKER,), jnp.int32),
            pltpu.VMEM((NBUF, CHUNK, 2 * EMB), jnp.float32),
            pltpu.VMEM((NBUF, EMB, _PITCH), jnp.float32),
            pltpu.VMEM((SEQ_LEN, EMB), jnp.float32),
            [pltpu.SemaphoreType.DMA] * NBUF,
            [pltpu.SemaphoreType.DMA] * NBUF,
        ],
        compiler_params=pltpu.CompilerParams(
            use_tc_tiling_on_sc=False, needs_layout_passes=False),
    )
    def emb_kernel(idx_hbm, wt_hbm, pos_hbm, out_hbm,
                   idx_v, rows_v, outt_v, pos_v, g_sems, o_sems):
        wid = lax.axis_index("s") * NUM_CORES + lax.axis_index("c")
        base = wid * PER_WORKER
        kbase = wid * CHUNKS_PER_WORKER

        pltpu.sync_copy(idx_hbm.at[pl.ds(base, PER_WORKER)], idx_v)
        pltpu.sync_copy(pos_hbm, pos_v)

        def start_gather(g, b):
            pltpu.async_copy(wt_hbm.at[idx_v.at[pl.ds(g * CHUNK, CHUNK)]],
                             rows_v.at[b], g_sems[b])

        def writeback(b, l, bg, issue):
            for cg in range(_CG):
                cp = pltpu.make_async_copy(
                    outt_v.at[b, pl.ds(cg * 8, 8), pl.ds(0, CHUNK)],
                    out_hbm.at[l, cg, bg], o_sems[b])
                if issue:
                    cp.start()
                else:
                    cp.wait()

        for b in range(PREFETCH):
            start_gather(b, b)

        @pl.loop(0, CHUNKS_PER_WORKER, step=NBUF)
        def _block(k):
            for b in range(NBUF):
                g = k + b
                kk = kbase + g
                l = kk // BG
                bg = lax.rem(kk, BG)

                pltpu.make_async_copy(
                    wt_hbm.at[idx_v.at[pl.ds(g * CHUNK, CHUNK)]],
                    rows_v.at[b], g_sems[b]).wait()

                @pl.when(g >= NBUF)
                def _():
                    kp = kbase + g - NBUF
                    writeback(b, kp // BG, lax.rem(kp, BG), issue=False)

                outt2d = outt_v.at[b]
                iota = lax.iota(jnp.int32, _LANE)
                pvs = [pos_v[l, pl.ds(s * _LANE, _LANE)] for s in range(4)]

                @pl.loop(0, CHUNK, unroll=8)
                def _row(j):
                    jidx = jnp.full((_LANE,), j, dtype=jnp.int32)
                    for s in range(4):
                        sl = pl.ds(s * _LANE, _LANE)
                        vec = rows_v[b, j, sl] + pvs[s]
                        plsc.store_scatter(outt2d,
                                           [iota + s * _LANE, jidx], vec)

                writeback(b, l, bg, issue=True)

                t = g + PREFETCH

                @pl.when(t < CHUNKS_PER_WORKER)
                def _():
                    start_gather(t, (b + PREFETCH) % NBUF)

        for g in range(CHUNKS_PER_WORKER - NBUF, CHUNKS_PER_WORKER):
            b = g % NBUF
            kk = kbase + g
            writeback(b, kk // BG, lax.rem(kk, BG), issue=False)

    return emb_kernel


def kernel(inputs, word_table, pos_table):
    idx_t_flat = inputs.T.reshape(TOTAL).astype(jnp.int32)
    wt_pad = jnp.pad(word_table, ((0, 0), (0, EMB)))
    out5d = _make_kernel()(idx_t_flat, wt_pad, pos_table)
    return out5d.transpose((2, 4, 0, 1, 3)).reshape(BATCH, SEQ_LEN, EMB)


# speedup vs baseline: 2.9446x; 1.3479x over previous
import functools

import jax
import jax.numpy as jnp
from jax import lax
from jax.experimental import pallas as pl
from jax.experimental.pallas import tpu as pltpu
from jax.experimental.pallas import tpu_sc as plsc

BATCH = 4096
SEQ_LEN = 200
EMB = 64

NUM_CORES = 2
NUM_SUBCORES = 16
NUM_WORKERS = NUM_CORES * NUM_SUBCORES

TOTAL = BATCH * SEQ_LEN
PER_WORKER = TOTAL // NUM_WORKERS
CHUNK = 128
CHUNKS_PER_WORKER = PER_WORKER // CHUNK
BG = BATCH // CHUNK
NBUF = 2
PREFETCH = 2

_LANE = 16
_CG = EMB // 8
_PITCH = CHUNK + 1


def _make_kernel():
    mesh = plsc.VectorSubcoreMesh(core_axis_name="c", subcore_axis_name="s")

    @functools.partial(
        pl.kernel,
        out_type=jax.ShapeDtypeStruct((SEQ_LEN, _CG, BG, 8, CHUNK), jnp.float32),
        mesh=mesh,
        scratch_types=[
            pltpu.VMEM((PER_WORKER,), jnp.int32),
            pltpu.VMEM((NBUF, CHUNK, 2 * EMB), jnp.float32),
            pltpu.VMEM((NBUF, EMB, _PITCH), jnp.float32),
            pltpu.VMEM((SEQ_LEN, EMB), jnp.float32),
            [pltpu.SemaphoreType.DMA] * NBUF,
            [pltpu.SemaphoreType.DMA] * NBUF,
        ],
        compiler_params=pltpu.CompilerParams(
            use_tc_tiling_on_sc=False, needs_layout_passes=False),
    )
    def emb_kernel(idx_hbm, wt_hbm, pos_hbm, out_hbm,
                   idx_v, rows_v, outt_v, pos_v, g_sems, o_sems):
        wid = lax.axis_index("s") * NUM_CORES + lax.axis_index("c")
        base = wid * PER_WORKER
        kbase = wid * CHUNKS_PER_WORKER

        pltpu.sync_copy(idx_hbm.at[pl.ds(base, PER_WORKER)], idx_v)
        pltpu.sync_copy(pos_hbm, pos_v)

        def start_gather(g, b):
            pltpu.async_copy(wt_hbm.at[idx_v.at[pl.ds(g * CHUNK, CHUNK)]],
                             rows_v.at[b], g_sems[b])

        def writeback(b, l, bg, issue):
            for cg in range(_CG):
                cp = pltpu.make_async_copy(
                    outt_v.at[b, pl.ds(cg * 8, 8), pl.ds(0, CHUNK)],
                    out_hbm.at[l, cg, bg], o_sems[b])
                if issue:
                    cp.start()
                else:
                    cp.wait()

        for b in range(PREFETCH):
            start_gather(b, b)

        @pl.loop(0, CHUNKS_PER_WORKER, step=NBUF)
        def _block(k):
            for b in range(NBUF):
                g = k + b
                kk = kbase + g
                l = kk // BG
                bg = lax.rem(kk, BG)

                pltpu.make_async_copy(
                    wt_hbm.at[idx_v.at[pl.ds(g * CHUNK, CHUNK)]],
                    rows_v.at[b], g_sems[b]).wait()

                @pl.when(g >= NBUF)
                def _():
                    kp = kbase + g - NBUF
                    writeback(b, kp // BG, lax.rem(kp, BG), issue=False)

                outt2d = outt_v.at[b]
                iota = lax.iota(jnp.int32, _LANE)
                pvs = [pos_v[l, pl.ds(s * _LANE, _LANE)] for s in range(4)]

                @functools.partial(plsc.parallel_loop, 0, CHUNK, unroll=4)
                def _row(j):
                    jidx = jnp.full((_LANE,), j, dtype=jnp.int32)
                    for s in range(4):
                        sl = pl.ds(s * _LANE, _LANE)
                        vec = rows_v[b, j, sl] + pvs[s]
                        plsc.store_scatter(outt2d,
                                           [iota + s * _LANE, jidx], vec)

                writeback(b, l, bg, issue=True)

                t = g + PREFETCH

                @pl.when(t < CHUNKS_PER_WORKER)
                def _():
                    start_gather(t, (b + PREFETCH) % NBUF)

        for g in range(CHUNKS_PER_WORKER - NBUF, CHUNKS_PER_WORKER):
            b = g % NBUF
            kk = kbase + g
            writeback(b, kk // BG, lax.rem(kk, BG), issue=False)

    return emb_kernel


def kernel(inputs, word_table, pos_table):
    idx_t_flat = inputs.T.reshape(TOTAL).astype(jnp.int32)
    wt_pad = jnp.pad(word_table, ((0, 0), (0, EMB)))
    out5d = _make_kernel()(idx_t_flat, wt_pad, pos_table)
    return out5d.transpose((2, 4, 0, 1, 3)).reshape(BATCH, SEQ_LEN, EMB)
